# Initial kernel scaffold; baseline (speedup 1.0000x reference)
import jax
import jax.numpy as jnp
from jax.experimental import pallas as pl


def kernel(q, k, v, offset, Wq, bq, Wk, bk, Wv, bv, W1, b1, W2, b2):
    raise NotImplementedError("write your pallas kernel here")


# speedup vs baseline: 7.1072x; 7.1072x over previous
import functools

import jax
import jax.numpy as jnp
from jax import lax
from jax.experimental import pallas as pl
from jax.experimental.pallas import tpu as pltpu
from jax.experimental.pallas import tpu_sc as plsc

B, T, C, H, W = 1, 2, 96, 128, 128
KH, KW = 3, 3
G = 6
CG = C // G
K = KH * KW
HW = H * W
L = T * K

NW = 32
PX = 32
PPW = HW // NW
NCHUNK = PPW // PX
NCH = T * G * K * 2


def _proj_body(x_ref, w_ref, b_ref, o_ref):
    x = x_ref[0]
    w = w_ref[0]
    y = lax.dot_general(x, w, (((0,), (0,)), ((), ())),
                        preferred_element_type=jnp.float32)
    o_ref[0] = y + b_ref[0]


def _project(x5, wt5, b5):
    BK = 2048
    return pl.pallas_call(
        _proj_body,
        grid=(5, HW // BK),
        in_specs=[
            pl.BlockSpec((1, C, BK), lambda i, j: (i, 0, j)),
            pl.BlockSpec((1, C, C), lambda i, j: (i, 0, 0)),
            pl.BlockSpec((1, 1, C), lambda i, j: (i, 0, 0)),
        ],
        out_specs=pl.BlockSpec((1, BK, C), lambda i, j: (i, j, 0)),
        out_shape=jax.ShapeDtypeStruct((5, HW, C), jnp.float32),
    )(x5, wt5, b5.reshape(5, 1, C))


def _ffn_body(x_ref, w1_ref, b1_ref, w2_ref, b2_ref, o_ref):
    x = x_ref[...]
    h = jax.nn.gelu(jnp.dot(x, w1_ref[...], preferred_element_type=jnp.float32)
                    + b1_ref[...][None, :])
    o_ref[...] = (jnp.dot(h, w2_ref[...], preferred_element_type=jnp.float32)
                  + b2_ref[...][None, :] + x)


def _ffn(x, w1t, b1, w2t, b2):
    BK = 2048
    return pl.pallas_call(
        _ffn_body,
        grid=(HW // BK,),
        in_specs=[
            pl.BlockSpec((BK, C), lambda i: (i, 0)),
            pl.BlockSpec((C, 2 * C), lambda i: (0, 0)),
            pl.BlockSpec((2 * C,), lambda i: (0,)),
            pl.BlockSpec((2 * C, C), lambda i: (0, 0)),
            pl.BlockSpec((C,), lambda i: (0,)),
        ],
        out_specs=pl.BlockSpec((BK, C), lambda i: (i, 0)),
        out_shape=jax.ShapeDtypeStruct((HW, C), jnp.float32),
    )(x, w1t, b1, w2t, b2)


def _floor_f32(x):
    ti = x.astype(jnp.int32).astype(jnp.float32)
    return ti - jnp.where(x < ti, 1.0, 0.0)


def _attn_body(ktab, vtab, qtab, offc, out_hbm,
               offv, qv, idxv, wv, rows, logit, outv, sem):
    wid = lax.axis_index("s") * 2 + lax.axis_index("c")
    lanes = lax.iota(jnp.int32, 16)

    def chunk_body(ci, _):
        p0 = wid * PPW + ci * PX
        pltpu.sync_copy(offc.at[wid * NCHUNK + ci], offv)
        pltpu.sync_copy(qtab.at[pl.ds(p0 * G, PX * G)], qv)

        def zero_body(i, _):
            outv.at[i][...] = jnp.zeros((16,), jnp.float32)
            return 0
        lax.fori_loop(0, PX * G, zero_body, 0)

        def idx_body(s, _):
            tg = s // (K * 2)
            r = s - tg * (K * 2)
            k = r // 2
            half = r - k * 2
            t = tg // G
            g = tg - t * G
            ki = k // KW
            kj = k - ki * KW
            p_idx = p0 + half * 16 + lanes
            pyv = p_idx >> 7
            pxv = p_idx & 127
            dy = offv[tg * (K * 2) + k * 2, pl.ds(half * 16, 16)]
            dx = offv[tg * (K * 2) + k * 2 + 1, pl.ds(half * 16, 16)]
            sy = pyv.astype(jnp.float32) + (ki - 1).astype(jnp.float32) + dy
            sx = pxv.astype(jnp.float32) + (kj - 1).astype(jnp.float32) + dx
            sy = jnp.minimum(jnp.maximum(sy, -2.0), 130.0)
            sx = jnp.minimum(jnp.maximum(sx, -2.0), 130.0)
            y0 = _floor_f32(sy)
            x0 = _floor_f32(sx)
            wy1 = sy - y0
            wx1 = sx - x0
            for a in range(2):
                ya = y0 + float(a)
                vay = (ya >= 0.0) & (ya <= float(H - 1))
                yi = jnp.minimum(jnp.maximum(ya, 0.0), float(H - 1)).astype(jnp.int32)
                wya = wy1 if a == 1 else 1.0 - wy1
                for bq_ in range(2):
                    xb = x0 + float(bq_)
                    vbx = (xb >= 0.0) & (xb <= float(W - 1))
                    xi = jnp.minimum(jnp.maximum(xb, 0.0), float(W - 1)).astype(jnp.int32)
                    wxb = wx1 if bq_ == 1 else 1.0 - wx1
                    c = a * 2 + bq_
                    msk = jnp.where(vay & vbx, 1.0, 0.0)
                    ridx = t * (HW * G) + ((yi << 7) + xi) * G + g
                    idxv.at[tg, k][pl.ds((c * 2 + half) * 16, 16)] = ridx
                    wv.at[tg * (K * 8) + k * 8 + c * 2 + half][...] = wya * wxb * msk
            return 0
        lax.fori_loop(0, T * G * K * 2, idx_body, 0)

        def logits_tg(tg, _):
            t = tg // G
            g = tg - t * G
            cps = [pltpu.async_copy(ktab.at[idxv.at[tg, k]],
                                    rows.at[pl.ds(k * 128, 128)], sem)
                   for k in range(K)]
            for cp in cps:
                cp.wait()

            def k_loop(k, _):
                wb = tg * (K * 8) + k * 8
                for half in range(2):
                    w0 = wv[wb + half]
                    w1 = wv[wb + 2 + half]
                    w2 = wv[wb + 4 + half]
                    w3 = wv[wb + 6 + half]

                    def lane_loop(ln, lg):
                        base = k * 128 + half * 16 + ln
                        spl = jnp.full((16,), ln, jnp.int32)
                        ks = (w0[spl] * rows[base]
                              + w1[spl] * rows[base + 32]
                              + w2[spl] * rows[base + 64]
                              + w3[spl] * rows[base + 96])
                        u = ks * qv[(half * 16 + ln) * G + g]
                        for stp in (8, 4, 2, 1):
                            u = u + u[(lanes + stp) & 15]
                        return jnp.where(lanes == ln, u, lg)
                    lg = lax.fori_loop(0, 16, lane_loop,
                                       jnp.zeros((16,), jnp.float32))
                    logit.at[t * K + k, g][pl.ds(half * 16, 16)] = lg
                return 0
            lax.fori_loop(0, K, k_loop, 0)
            return 0
        lax.fori_loop(0, T * G, logits_tg, 0)

        for g in range(G):
            for half in range(2):
                sl = pl.ds(half * 16, 16)
                vals = [logit[l, g, sl] for l in range(L)]
                m = vals[0]
                for l in range(1, L):
                    m = jnp.maximum(m, vals[l])
                es = [jnp.exp(v - m) for v in vals]
                ssum = es[0]
                for l in range(1, L):
                    ssum = ssum + es[l]
                inv = 1.0 / ssum
                for l in range(L):
                    logit.at[l, g][sl] = es[l] * inv

        def acc_tg(tg, _):
            t = tg // G
            g = tg - t * G
            cps = [pltpu.async_copy(vtab.at[idxv.at[tg, k]],
                                    rows.at[pl.ds(k * 128, 128)], sem)
                   for k in range(K)]
            for cp in cps:
                cp.wait()

            def k_loop(k, _):
                wb = tg * (K * 8) + k * 8
                for half in range(2):
                    w0 = wv[wb + half]
                    w1 = wv[wb + 2 + half]
                    w2 = wv[wb + 4 + half]
                    w3 = wv[wb + 6 + half]
                    awrow = logit[t * K + k, g, pl.ds(half * 16, 16)]

                    def lane_loop(ln, _):
                        base = k * 128 + half * 16 + ln
                        spl = jnp.full((16,), ln, jnp.int32)
                        vs = (w0[spl] * rows[base]
                              + w1[spl] * rows[base + 32]
                              + w2[spl] * rows[base + 64]
                              + w3[spl] * rows[base + 96])
                        orow = (half * 16 + ln) * G + g
                        outv.at[orow][...] = outv[orow] + awrow[spl] * vs
                        return 0
                    lax.fori_loop(0, 16, lane_loop, 0)
                return 0
            lax.fori_loop(0, K, k_loop, 0)
            return 0
        lax.fori_loop(0, T * G, acc_tg, 0)

        pltpu.sync_copy(outv, out_hbm.at[pl.ds(p0 * G, PX * G)])
        return 0

    lax.fori_loop(0, NCHUNK, chunk_body, 0)


def _sc_attention(ktab, vtab, qtab, offc):
    mesh = plsc.VectorSubcoreMesh(core_axis_name="c", subcore_axis_name="s")
    fn = functools.partial(
        pl.kernel,
        mesh=mesh,
        compiler_params=pltpu.CompilerParams(use_tc_tiling_on_sc=False),
        out_type=jax.ShapeDtypeStruct((HW * G, CG), jnp.float32),
        scratch_types=[
            pltpu.VMEM((NCH, PX), jnp.float32),
            pltpu.VMEM((PX * G, CG), jnp.float32),
            pltpu.VMEM((T * G, K, 128), jnp.int32),
            pltpu.VMEM((T * G * K * 8, CG), jnp.float32),
            pltpu.VMEM((K * 128, CG), jnp.float32),
            pltpu.VMEM((L, G, PX), jnp.float32),
            pltpu.VMEM((PX * G, CG), jnp.float32),
            pltpu.SemaphoreType.DMA,
        ],
    )(_attn_body)
    return fn(ktab, vtab, qtab, offc)


def kernel(q, k, v, offset, Wq, bq, Wk, bk, Wv, bv, W1, b1, W2, b2):
    scale = 1.0 / jnp.sqrt(jnp.float32(CG))
    x5 = jnp.concatenate([q.reshape(1, C, HW),
                          k.reshape(T, C, HW),
                          v.reshape(T, C, HW)], axis=0)
    wt5 = jnp.stack([Wq.T * scale, Wk.T, Wk.T, Wv.T, Wv.T], axis=0)
    b5 = jnp.stack([bq * scale, bk, bk, bv, bv], axis=0)
    y5 = _project(x5, wt5, b5)

    qtab = y5[0].reshape(HW * G, CG)
    ktab = y5[1:3].reshape(T * HW * G, CG)
    vtab = y5[3:5].reshape(T * HW * G, CG)
    offc = (offset.reshape(NCH, HW // PX, PX)
            .transpose(1, 0, 2))

    att = _sc_attention(ktab, vtab, qtab, offc)
    att = att.reshape(HW, C)

    y = _ffn(att, W1.T, b1, W2.T, b2)
    return y.T.reshape(B, 1, C, H, W)


# speedup vs baseline: 10.2400x; 1.4408x over previous
import functools

import jax
import jax.numpy as jnp
from jax import lax
from jax.experimental import pallas as pl
from jax.experimental.pallas import tpu as pltpu
from jax.experimental.pallas import tpu_sc as plsc

B, T, C, H, W = 1, 2, 96, 128, 128
KH, KW = 3, 3
G = 6
CG = C // G
K = KH * KW
HW = H * W
L = T * K

NW = 32
PX = 32
PPW = HW // NW
NCHUNK = PPW // PX
NCH = T * G * K * 2


def _proj_body(x_ref, w_ref, b_ref, o_ref):
    x = x_ref[0]
    w = w_ref[0]
    y = lax.dot_general(x, w, (((0,), (0,)), ((), ())),
                        preferred_element_type=jnp.float32)
    o_ref[0] = y + b_ref[0]


def _project(x5, wt5, b5):
    BK = 2048
    return pl.pallas_call(
        _proj_body,
        grid=(5, HW // BK),
        in_specs=[
            pl.BlockSpec((1, C, BK), lambda i, j: (i, 0, j)),
            pl.BlockSpec((1, C, C), lambda i, j: (i, 0, 0)),
            pl.BlockSpec((1, 1, C), lambda i, j: (i, 0, 0)),
        ],
        out_specs=pl.BlockSpec((1, BK, C), lambda i, j: (i, j, 0)),
        out_shape=jax.ShapeDtypeStruct((5, HW, C), jnp.float32),
    )(x5, wt5, b5.reshape(5, 1, C))


def _ffn_body(x_ref, w1_ref, b1_ref, w2_ref, b2_ref, o_ref):
    x = x_ref[...]
    h = jax.nn.gelu(jnp.dot(x, w1_ref[...], preferred_element_type=jnp.float32)
                    + b1_ref[...][None, :])
    o_ref[...] = (jnp.dot(h, w2_ref[...], preferred_element_type=jnp.float32)
                  + b2_ref[...][None, :] + x)


def _ffn(x, w1t, b1, w2t, b2):
    BK = 2048
    return pl.pallas_call(
        _ffn_body,
        grid=(HW // BK,),
        in_specs=[
            pl.BlockSpec((BK, C), lambda i: (i, 0)),
            pl.BlockSpec((C, 2 * C), lambda i: (0, 0)),
            pl.BlockSpec((2 * C,), lambda i: (0,)),
            pl.BlockSpec((2 * C, C), lambda i: (0, 0)),
            pl.BlockSpec((C,), lambda i: (0,)),
        ],
        out_specs=pl.BlockSpec((BK, C), lambda i: (i, 0)),
        out_shape=jax.ShapeDtypeStruct((HW, C), jnp.float32),
    )(x, w1t, b1, w2t, b2)


def _floor_f32(x):
    ti = x.astype(jnp.int32).astype(jnp.float32)
    return ti - jnp.where(x < ti, 1.0, 0.0)


def _attn_body(ktab, vtab, qtab, offc, out_hbm,
               offv, qv, idxv, wv, rows, logit, outv, sem):
    wid = lax.axis_index("s") * 2 + lax.axis_index("c")
    lanes = lax.iota(jnp.int32, 16)

    def chunk_body(ci, _):
        p0 = wid * PPW + ci * PX
        pltpu.sync_copy(offc.at[wid * NCHUNK + ci], offv)
        pltpu.sync_copy(qtab.at[pl.ds(p0 * G, PX * G)], qv)

        def zero_body(i, _):
            outv.at[i][...] = jnp.zeros((16,), jnp.float32)
            return 0
        lax.fori_loop(0, PX * G, zero_body, 0)

        def idx_body(s, _):
            tg = s // (K * 2)
            r = s - tg * (K * 2)
            k = r // 2
            half = r - k * 2
            t = tg // G
            g = tg - t * G
            ki = k // KW
            kj = k - ki * KW
            p_idx = p0 + half * 16 + lanes
            pyv = p_idx >> 7
            pxv = p_idx & 127
            dy = offv[tg * (K * 2) + k * 2, pl.ds(half * 16, 16)]
            dx = offv[tg * (K * 2) + k * 2 + 1, pl.ds(half * 16, 16)]
            sy = pyv.astype(jnp.float32) + (ki - 1).astype(jnp.float32) + dy
            sx = pxv.astype(jnp.float32) + (kj - 1).astype(jnp.float32) + dx
            sy = jnp.minimum(jnp.maximum(sy, -2.0), 130.0)
            sx = jnp.minimum(jnp.maximum(sx, -2.0), 130.0)
            y0 = _floor_f32(sy)
            x0 = _floor_f32(sx)
            wy1 = sy - y0
            wx1 = sx - x0
            for a in range(2):
                ya = y0 + float(a)
                vay = (ya >= 0.0) & (ya <= float(H - 1))
                yi = jnp.minimum(jnp.maximum(ya, 0.0), float(H - 1)).astype(jnp.int32)
                wya = wy1 if a == 1 else 1.0 - wy1
                for bq_ in range(2):
                    xb = x0 + float(bq_)
                    vbx = (xb >= 0.0) & (xb <= float(W - 1))
                    xi = jnp.minimum(jnp.maximum(xb, 0.0), float(W - 1)).astype(jnp.int32)
                    wxb = wx1 if bq_ == 1 else 1.0 - wx1
                    c = a * 2 + bq_
                    msk = jnp.where(vay & vbx, 1.0, 0.0)
                    ridx = t * (HW * G) + ((yi << 7) + xi) * G + g
                    idxv.at[tg, k][pl.ds((c * 2 + half) * 16, 16)] = ridx
                    wv.at[tg * (K * 8) + k * 8 + c * 2 + half][...] = wya * wxb * msk
            return 0
        lax.fori_loop(0, T * G * K * 2, idx_body, 0)

        def fire(tab, tg, buf):
            for k in range(K):
                pltpu.async_copy(tab.at[idxv.at[tg, k]],
                                 rows.at[buf, pl.ds(k * 128, 128)], sem)

        def drain(tab, tg, buf):
            for k in range(K):
                pltpu.make_async_copy(tab.at[idxv.at[tg, k]],
                                      rows.at[buf, pl.ds(k * 128, 128)],
                                      sem).wait()

        fire(ktab, 0, 0)

        def logits_tg(tg, _):
            t = tg // G
            g = tg - t * G
            buf = tg & 1
            drain(ktab, tg, buf)

            @pl.when(tg < T * G - 1)
            def _():
                fire(ktab, tg + 1, 1 - buf)

            def k_loop(k, _):
                wb = tg * (K * 8) + k * 8
                for half in range(2):
                    w0 = wv[wb + half]
                    w1 = wv[wb + 2 + half]
                    w2 = wv[wb + 4 + half]
                    w3 = wv[wb + 6 + half]
                    lg = jnp.zeros((16,), jnp.float32)
                    for ln in range(16):
                        base = k * 128 + half * 16 + ln
                        spl = jnp.full((16,), ln, jnp.int32)
                        ks = (w0[spl] * rows[buf, base]
                              + w1[spl] * rows[buf, base + 32]
                              + w2[spl] * rows[buf, base + 64]
                              + w3[spl] * rows[buf, base + 96])
                        u = ks * qv[(half * 16 + ln) * G + g]
                        for stp in (8, 4, 2, 1):
                            u = u + u[(lanes + stp) & 15]
                        lg = jnp.where(lanes == ln, u, lg)
                    logit.at[t * K + k, g][pl.ds(half * 16, 16)] = lg
                return 0
            lax.fori_loop(0, K, k_loop, 0)
            return 0
        lax.fori_loop(0, T * G, logits_tg, 0)
        fire(vtab, 0, 0)

        for g in range(G):
            for half in range(2):
                sl = pl.ds(half * 16, 16)
                vals = [logit[l, g, sl] for l in range(L)]
                m = vals[0]
                for l in range(1, L):
                    m = jnp.maximum(m, vals[l])
                es = [jnp.exp(v - m) for v in vals]
                ssum = es[0]
                for l in range(1, L):
                    ssum = ssum + es[l]
                inv = 1.0 / ssum
                for l in range(L):
                    logit.at[l, g][sl] = es[l] * inv

        def acc_tg(tg, _):
            t = tg // G
            g = tg - t * G
            buf = tg & 1
            drain(vtab, tg, buf)

            @pl.when(tg < T * G - 1)
            def _():
                fire(vtab, tg + 1, 1 - buf)

            def k_loop(k, _):
                wb = tg * (K * 8) + k * 8
                for half in range(2):
                    w0 = wv[wb + half]
                    w1 = wv[wb + 2 + half]
                    w2 = wv[wb + 4 + half]
                    w3 = wv[wb + 6 + half]
                    awrow = logit[t * K + k, g, pl.ds(half * 16, 16)]
                    for ln in range(16):
                        base = k * 128 + half * 16 + ln
                        spl = jnp.full((16,), ln, jnp.int32)
                        vs = (w0[spl] * rows[buf, base]
                              + w1[spl] * rows[buf, base + 32]
                              + w2[spl] * rows[buf, base + 64]
                              + w3[spl] * rows[buf, base + 96])
                        orow = (half * 16 + ln) * G + g
                        outv.at[orow][...] = outv[orow] + awrow[spl] * vs
                return 0
            lax.fori_loop(0, K, k_loop, 0)
            return 0
        lax.fori_loop(0, T * G, acc_tg, 0)

        pltpu.sync_copy(outv, out_hbm.at[pl.ds(p0 * G, PX * G)])
        return 0

    lax.fori_loop(0, NCHUNK, chunk_body, 0)


def _sc_attention(ktab, vtab, qtab, offc):
    mesh = plsc.VectorSubcoreMesh(core_axis_name="c", subcore_axis_name="s")
    fn = functools.partial(
        pl.kernel,
        mesh=mesh,
        compiler_params=pltpu.CompilerParams(use_tc_tiling_on_sc=False),
        out_type=jax.ShapeDtypeStruct((HW * G, CG), jnp.float32),
        scratch_types=[
            pltpu.VMEM((NCH, PX), jnp.float32),
            pltpu.VMEM((PX * G, CG), jnp.float32),
            pltpu.VMEM((T * G, K, 128), jnp.int32),
            pltpu.VMEM((T * G * K * 8, CG), jnp.float32),
            pltpu.VMEM((2, K * 128, CG), jnp.float32),
            pltpu.VMEM((L, G, PX), jnp.float32),
            pltpu.VMEM((PX * G, CG), jnp.float32),
            pltpu.SemaphoreType.DMA,
        ],
    )(_attn_body)
    return fn(ktab, vtab, qtab, offc)


def kernel(q, k, v, offset, Wq, bq, Wk, bk, Wv, bv, W1, b1, W2, b2):
    scale = 1.0 / jnp.sqrt(jnp.float32(CG))
    x5 = jnp.concatenate([q.reshape(1, C, HW),
                          k.reshape(T, C, HW),
                          v.reshape(T, C, HW)], axis=0)
    wt5 = jnp.stack([Wq.T * scale, Wk.T, Wk.T, Wv.T, Wv.T], axis=0)
    b5 = jnp.stack([bq * scale, bk, bk, bv, bv], axis=0)
    y5 = _project(x5, wt5, b5)

    qtab = y5[0].reshape(HW * G, CG)
    ktab = y5[1:3].reshape(T * HW * G, CG)
    vtab = y5[3:5].reshape(T * HW * G, CG)
    offc = (offset.reshape(NCH, HW // PX, PX)
            .transpose(1, 0, 2))

    att = _sc_attention(ktab, vtab, qtab, offc)
    att = att.reshape(HW, C)

    y = _ffn(att, W1.T, b1, W2.T, b2)
    return y.T.reshape(B, 1, C, H, W)


# speedup vs baseline: 10.6733x; 1.0423x over previous
import functools

import jax
import jax.numpy as jnp
from jax import lax
from jax.experimental import pallas as pl
from jax.experimental.pallas import tpu as pltpu
from jax.experimental.pallas import tpu_sc as plsc

B, T, C, H, W = 1, 2, 96, 128, 128
KH, KW = 3, 3
G = 6
CG = C // G
K = KH * KW
HW = H * W
L = T * K

NW = 32
PX = 32
PPW = HW // NW
NCHUNK = PPW // PX
NCH = T * G * K * 2


def _proj_body(x_ref, w_ref, b_ref, o_ref):
    x = x_ref[0]
    w = w_ref[0]
    y = lax.dot_general(x, w, (((0,), (0,)), ((), ())),
                        preferred_element_type=jnp.float32)
    o_ref[0] = y + b_ref[0]


def _project(x5, wt5, b5):
    BK = 2048
    return pl.pallas_call(
        _proj_body,
        grid=(5, HW // BK),
        in_specs=[
            pl.BlockSpec((1, C, BK), lambda i, j: (i, 0, j)),
            pl.BlockSpec((1, C, C), lambda i, j: (i, 0, 0)),
            pl.BlockSpec((1, 1, C), lambda i, j: (i, 0, 0)),
        ],
        out_specs=pl.BlockSpec((1, BK, C), lambda i, j: (i, j, 0)),
        out_shape=jax.ShapeDtypeStruct((5, HW, C), jnp.float32),
    )(x5, wt5, b5.reshape(5, 1, C))


def _ffn_body(x_ref, w1_ref, b1_ref, w2_ref, b2_ref, o_ref):
    x = x_ref[...]
    h = jax.nn.gelu(jnp.dot(x, w1_ref[...], preferred_element_type=jnp.float32)
                    + b1_ref[...][None, :])
    o_ref[...] = (jnp.dot(h, w2_ref[...], preferred_element_type=jnp.float32)
                  + b2_ref[...][None, :] + x)


def _ffn(x, w1t, b1, w2t, b2):
    BK = 2048
    return pl.pallas_call(
        _ffn_body,
        grid=(HW // BK,),
        in_specs=[
            pl.BlockSpec((BK, C), lambda i: (i, 0)),
            pl.BlockSpec((C, 2 * C), lambda i: (0, 0)),
            pl.BlockSpec((2 * C,), lambda i: (0,)),
            pl.BlockSpec((2 * C, C), lambda i: (0, 0)),
            pl.BlockSpec((C,), lambda i: (0,)),
        ],
        out_specs=pl.BlockSpec((BK, C), lambda i: (i, 0)),
        out_shape=jax.ShapeDtypeStruct((HW, C), jnp.float32),
    )(x, w1t, b1, w2t, b2)


def _floor_f32(x):
    ti = x.astype(jnp.int32).astype(jnp.float32)
    return ti - jnp.where(x < ti, 1.0, 0.0)


def _attn_body(ktab, vtab, qtab, offc, out_hbm,
               offv, qv, idxv, wv, rows, logit, outv, sem):
    wid = lax.axis_index("s") * 2 + lax.axis_index("c")
    lanes = lax.iota(jnp.int32, 16)

    def chunk_body(ci, _):
        p0 = wid * PPW + ci * PX
        pltpu.sync_copy(offc.at[wid * NCHUNK + ci], offv)
        pltpu.sync_copy(qtab.at[pl.ds(p0 * G, PX * G)], qv)

        def zero_body(i, _):
            outv.at[i][...] = jnp.zeros((16,), jnp.float32)
            return 0
        lax.fori_loop(0, PX * G, zero_body, 0)

        def idx_body(s, _):
            tg = s // (K * 2)
            r = s - tg * (K * 2)
            k = r // 2
            half = r - k * 2
            t = tg // G
            g = tg - t * G
            ki = k // KW
            kj = k - ki * KW
            p_idx = p0 + half * 16 + lanes
            pyv = p_idx >> 7
            pxv = p_idx & 127
            dy = offv[tg * (K * 2) + k * 2, pl.ds(half * 16, 16)]
            dx = offv[tg * (K * 2) + k * 2 + 1, pl.ds(half * 16, 16)]
            sy = pyv.astype(jnp.float32) + (ki - 1).astype(jnp.float32) + dy
            sx = pxv.astype(jnp.float32) + (kj - 1).astype(jnp.float32) + dx
            sy = jnp.minimum(jnp.maximum(sy, -2.0), 130.0)
            sx = jnp.minimum(jnp.maximum(sx, -2.0), 130.0)
            y0 = _floor_f32(sy)
            x0 = _floor_f32(sx)
            wy1 = sy - y0
            wx1 = sx - x0
            for a in range(2):
                ya = y0 + float(a)
                vay = (ya >= 0.0) & (ya <= float(H - 1))
                yi = jnp.minimum(jnp.maximum(ya, 0.0), float(H - 1)).astype(jnp.int32)
                wya = wy1 if a == 1 else 1.0 - wy1
                for bq_ in range(2):
                    xb = x0 + float(bq_)
                    vbx = (xb >= 0.0) & (xb <= float(W - 1))
                    xi = jnp.minimum(jnp.maximum(xb, 0.0), float(W - 1)).astype(jnp.int32)
                    wxb = wx1 if bq_ == 1 else 1.0 - wx1
                    c = a * 2 + bq_
                    msk = jnp.where(vay & vbx, 1.0, 0.0)
                    ridx = t * (HW * G) + ((yi << 7) + xi) * G + g
                    idxv.at[tg, k][pl.ds((c * 2 + half) * 16, 16)] = ridx
                    wv.at[tg * (K * 8) + k * 8 + c * 2 + half][...] = wya * wxb * msk
            return 0
        lax.fori_loop(0, T * G * K * 2, idx_body, 0)

        def fire(tab, tg, buf):
            for k in range(K):
                pltpu.async_copy(tab.at[idxv.at[tg, k]],
                                 rows.at[buf, pl.ds(k * 128, 128)], sem)

        def drain(tab, tg, buf):
            for k in range(K):
                pltpu.make_async_copy(tab.at[idxv.at[tg, k]],
                                      rows.at[buf, pl.ds(k * 128, 128)],
                                      sem).wait()

        fire(ktab, 0, 0)

        def logits_tg(tg, _):
            t = tg // G
            g = tg - t * G
            buf = tg & 1
            drain(ktab, tg, buf)

            @pl.when(tg < T * G - 1)
            def _():
                fire(ktab, tg + 1, 1 - buf)

            def k_loop(k, _):
                wb = tg * (K * 8) + k * 8
                for half in range(2):
                    w0 = wv[wb + half]
                    w1 = wv[wb + 2 + half]
                    w2 = wv[wb + 4 + half]
                    w3 = wv[wb + 6 + half]
                    ms = []
                    for ln in range(16):
                        base = k * 128 + half * 16 + ln
                        spl = jnp.full((16,), ln, jnp.int32)
                        ks = (w0[spl] * rows[buf, base]
                              + w1[spl] * rows[buf, base + 32]
                              + w2[spl] * rows[buf, base + 64]
                              + w3[spl] * rows[buf, base + 96])
                        ms.append(ks * qv[(half * 16 + ln) * G + g])
                    for s in (8, 4, 2, 1):
                        xp = lanes ^ s
                        mk = (lanes & s) == 0
                        h = len(ms) // 2
                        ms = [jnp.where(mk, ms[i], ms[i + h][xp])
                              + jnp.where(mk, ms[i][xp], ms[i + h])
                              for i in range(h)]
                    logit.at[t * K + k, g][pl.ds(half * 16, 16)] = ms[0]
                return 0
            lax.fori_loop(0, K, k_loop, 0)
            return 0
        lax.fori_loop(0, T * G, logits_tg, 0)
        fire(vtab, 0, 0)

        for g in range(G):
            for half in range(2):
                sl = pl.ds(half * 16, 16)
                vals = [logit[l, g, sl] for l in range(L)]
                m = vals[0]
                for l in range(1, L):
                    m = jnp.maximum(m, vals[l])
                es = [jnp.exp(v - m) for v in vals]
                ssum = es[0]
                for l in range(1, L):
                    ssum = ssum + es[l]
                inv = 1.0 / ssum
                for l in range(L):
                    logit.at[l, g][sl] = es[l] * inv

        def acc_tg(tg, _):
            t = tg // G
            g = tg - t * G
            buf = tg & 1
            drain(vtab, tg, buf)

            @pl.when(tg < T * G - 1)
            def _():
                fire(vtab, tg + 1, 1 - buf)

            def k_loop(k, _):
                wb = tg * (K * 8) + k * 8
                for half in range(2):
                    awrow = logit[t * K + k, g, pl.ds(half * 16, 16)]
                    w0 = wv[wb + half] * awrow
                    w1 = wv[wb + 2 + half] * awrow
                    w2 = wv[wb + 4 + half] * awrow
                    w3 = wv[wb + 6 + half] * awrow
                    for ln in range(16):
                        base = k * 128 + half * 16 + ln
                        spl = jnp.full((16,), ln, jnp.int32)
                        vs = (w0[spl] * rows[buf, base]
                              + w1[spl] * rows[buf, base + 32]
                              + w2[spl] * rows[buf, base + 64]
                              + w3[spl] * rows[buf, base + 96])
                        orow = (half * 16 + ln) * G + g
                        outv.at[orow][...] = outv[orow] + vs
                return 0
            lax.fori_loop(0, K, k_loop, 0)
            return 0
        lax.fori_loop(0, T * G, acc_tg, 0)

        pltpu.sync_copy(outv, out_hbm.at[pl.ds(p0 * G, PX * G)])
        return 0

    lax.fori_loop(0, NCHUNK, chunk_body, 0)


def _sc_attention(ktab, vtab, qtab, offc):
    mesh = plsc.VectorSubcoreMesh(core_axis_name="c", subcore_axis_name="s")
    fn = functools.partial(
        pl.kernel,
        mesh=mesh,
        compiler_params=pltpu.CompilerParams(use_tc_tiling_on_sc=False),
        out_type=jax.ShapeDtypeStruct((HW * G, CG), jnp.float32),
        scratch_types=[
            pltpu.VMEM((NCH, PX), jnp.float32),
            pltpu.VMEM((PX * G, CG), jnp.float32),
            pltpu.VMEM((T * G, K, 128), jnp.int32),
            pltpu.VMEM((T * G * K * 8, CG), jnp.float32),
            pltpu.VMEM((2, K * 128, CG), jnp.float32),
            pltpu.VMEM((L, G, PX), jnp.float32),
            pltpu.VMEM((PX * G, CG), jnp.float32),
            pltpu.SemaphoreType.DMA,
        ],
    )(_attn_body)
    return fn(ktab, vtab, qtab, offc)


def kernel(q, k, v, offset, Wq, bq, Wk, bk, Wv, bv, W1, b1, W2, b2):
    scale = 1.0 / jnp.sqrt(jnp.float32(CG))
    x5 = jnp.concatenate([q.reshape(1, C, HW),
                          k.reshape(T, C, HW),
                          v.reshape(T, C, HW)], axis=0)
    wt5 = jnp.stack([Wq.T * scale, Wk.T, Wk.T, Wv.T, Wv.T], axis=0)
    b5 = jnp.stack([bq * scale, bk, bk, bv, bv], axis=0)
    y5 = _project(x5, wt5, b5)

    qtab = y5[0].reshape(HW * G, CG)
    ktab = y5[1:3].reshape(T * HW * G, CG)
    vtab = y5[3:5].reshape(T * HW * G, CG)
    offc = (offset.reshape(NCH, HW // PX, PX)
            .transpose(1, 0, 2))

    att = _sc_attention(ktab, vtab, qtab, offc)
    att = att.reshape(HW, C)

    y = _ffn(att, W1.T, b1, W2.T, b2)
    return y.T.reshape(B, 1, C, H, W)


# speedup vs baseline: 10.9153x; 1.0227x over previous
import functools

import jax
import jax.numpy as jnp
from jax import lax
from jax.experimental import pallas as pl
from jax.experimental.pallas import tpu as pltpu
from jax.experimental.pallas import tpu_sc as plsc

B, T, C, H, W = 1, 2, 96, 128, 128
KH, KW = 3, 3
G = 6
CG = C // G
K = KH * KW
HW = H * W
L = T * K

NW = 32
PX = 32
PPW = HW // NW
NCHUNK = PPW // PX
NCH = T * G * K * 2


def _proj_body(x_ref, w_ref, b_ref, o_ref):
    x = x_ref[0]
    w = w_ref[0]
    y = lax.dot_general(x, w, (((0,), (0,)), ((), ())),
                        preferred_element_type=jnp.float32)
    o_ref[0] = y + b_ref[0]


def _project(x5, wt5, b5):
    BK = 2048
    return pl.pallas_call(
        _proj_body,
        grid=(5, HW // BK),
        in_specs=[
            pl.BlockSpec((1, C, BK), lambda i, j: (i, 0, j)),
            pl.BlockSpec((1, C, C), lambda i, j: (i, 0, 0)),
            pl.BlockSpec((1, 1, C), lambda i, j: (i, 0, 0)),
        ],
        out_specs=pl.BlockSpec((1, BK, C), lambda i, j: (i, j, 0)),
        out_shape=jax.ShapeDtypeStruct((5, HW, C), jnp.float32),
    )(x5, wt5, b5.reshape(5, 1, C))


def _ffn_body(x_ref, w1_ref, b1_ref, w2_ref, b2_ref, o_ref):
    x = x_ref[...]
    h = jax.nn.gelu(jnp.dot(x, w1_ref[...], preferred_element_type=jnp.float32)
                    + b1_ref[...][None, :])
    o_ref[...] = (jnp.dot(h, w2_ref[...], preferred_element_type=jnp.float32)
                  + b2_ref[...][None, :] + x)


def _ffn(x, w1t, b1, w2t, b2):
    BK = 2048
    return pl.pallas_call(
        _ffn_body,
        grid=(HW // BK,),
        in_specs=[
            pl.BlockSpec((BK, C), lambda i: (i, 0)),
            pl.BlockSpec((C, 2 * C), lambda i: (0, 0)),
            pl.BlockSpec((2 * C,), lambda i: (0,)),
            pl.BlockSpec((2 * C, C), lambda i: (0, 0)),
            pl.BlockSpec((C,), lambda i: (0,)),
        ],
        out_specs=pl.BlockSpec((BK, C), lambda i: (i, 0)),
        out_shape=jax.ShapeDtypeStruct((HW, C), jnp.float32),
    )(x, w1t, b1, w2t, b2)


def _floor_f32(x):
    ti = x.astype(jnp.int32).astype(jnp.float32)
    return ti - jnp.where(x < ti, 1.0, 0.0)


def _attn_body(ktab, vtab, qtab, offc, out_hbm,
               offv, qv, idxv, wv, rows, logit, outv, sem, sem2, sem3):
    wid = lax.axis_index("s") * 2 + lax.axis_index("c")
    lanes = lax.iota(jnp.int32, 16)

    pltpu.async_copy(offc.at[wid * NCHUNK], offv.at[0], sem2)
    pltpu.async_copy(qtab.at[pl.ds(wid * PPW * G, PX * G)], qv.at[0], sem2)

    def chunk_body(ci, _):
        p0 = wid * PPW + ci * PX
        cbuf = ci & 1
        pltpu.make_async_copy(offc.at[wid * NCHUNK + ci], offv.at[cbuf],
                              sem2).wait()
        pltpu.make_async_copy(qtab.at[pl.ds(p0 * G, PX * G)], qv.at[cbuf],
                              sem2).wait()

        @pl.when(ci < NCHUNK - 1)
        def _():
            pltpu.async_copy(offc.at[wid * NCHUNK + ci + 1],
                             offv.at[1 - cbuf], sem2)
            pltpu.async_copy(qtab.at[pl.ds((p0 + PX) * G, PX * G)],
                             qv.at[1 - cbuf], sem2)

        @pl.when(ci >= 2)
        def _():
            pltpu.make_async_copy(
                outv.at[cbuf],
                out_hbm.at[pl.ds((p0 - 2 * PX) * G, PX * G)], sem3).wait()

        def zero_body(i, _):
            outv.at[cbuf, i][...] = jnp.zeros((16,), jnp.float32)
            return 0
        lax.fori_loop(0, PX * G, zero_body, 0)

        def idx_body(s, _):
            tg = s // (K * 2)
            r = s - tg * (K * 2)
            k = r // 2
            half = r - k * 2
            t = tg // G
            g = tg - t * G
            ki = k // KW
            kj = k - ki * KW
            p_idx = p0 + half * 16 + lanes
            pyv = p_idx >> 7
            pxv = p_idx & 127
            dy = offv[cbuf, tg * (K * 2) + k * 2, pl.ds(half * 16, 16)]
            dx = offv[cbuf, tg * (K * 2) + k * 2 + 1, pl.ds(half * 16, 16)]
            sy = pyv.astype(jnp.float32) + (ki - 1).astype(jnp.float32) + dy
            sx = pxv.astype(jnp.float32) + (kj - 1).astype(jnp.float32) + dx
            sy = jnp.minimum(jnp.maximum(sy, -2.0), 130.0)
            sx = jnp.minimum(jnp.maximum(sx, -2.0), 130.0)
            y0 = _floor_f32(sy)
            x0 = _floor_f32(sx)
            wy1 = sy - y0
            wx1 = sx - x0
            for a in range(2):
                ya = y0 + float(a)
                vay = (ya >= 0.0) & (ya <= float(H - 1))
                yi = jnp.minimum(jnp.maximum(ya, 0.0), float(H - 1)).astype(jnp.int32)
                wya = wy1 if a == 1 else 1.0 - wy1
                for bq_ in range(2):
                    xb = x0 + float(bq_)
                    vbx = (xb >= 0.0) & (xb <= float(W - 1))
                    xi = jnp.minimum(jnp.maximum(xb, 0.0), float(W - 1)).astype(jnp.int32)
                    wxb = wx1 if bq_ == 1 else 1.0 - wx1
                    c = a * 2 + bq_
                    msk = jnp.where(vay & vbx, 1.0, 0.0)
                    ridx = t * (HW * G) + ((yi << 7) + xi) * G + g
                    idxv.at[tg, k][pl.ds((c * 2 + half) * 16, 16)] = ridx
                    wv.at[tg * (K * 8) + k * 8 + c * 2 + half][...] = wya * wxb * msk
            return 0
        lax.fori_loop(0, T * G * K * 2, idx_body, 0)

        def fire(tab, tg, buf):
            for k in range(K):
                pltpu.async_copy(tab.at[idxv.at[tg, k]],
                                 rows.at[buf, pl.ds(k * 128, 128)], sem)

        def drain(tab, tg, buf):
            for k in range(K):
                pltpu.make_async_copy(tab.at[idxv.at[tg, k]],
                                      rows.at[buf, pl.ds(k * 128, 128)],
                                      sem).wait()

        fire(ktab, 0, 0)

        def logits_tg(tg, _):
            t = tg // G
            g = tg - t * G
            buf = tg & 1
            drain(ktab, tg, buf)

            @pl.when(tg < T * G - 1)
            def _():
                fire(ktab, tg + 1, 1 - buf)

            def k_loop(k, _):
                wb = tg * (K * 8) + k * 8
                for half in range(2):
                    w0 = wv[wb + half]
                    w1 = wv[wb + 2 + half]
                    w2 = wv[wb + 4 + half]
                    w3 = wv[wb + 6 + half]
                    ms = []
                    for ln in range(16):
                        base = k * 128 + half * 16 + ln
                        spl = jnp.full((16,), ln, jnp.int32)
                        ks = (w0[spl] * rows[buf, base]
                              + w1[spl] * rows[buf, base + 32]
                              + w2[spl] * rows[buf, base + 64]
                              + w3[spl] * rows[buf, base + 96])
                        ms.append(ks * qv[cbuf, (half * 16 + ln) * G + g])
                    for s in (8, 4, 2, 1):
                        xp = lanes ^ s
                        mk = (lanes & s) == 0
                        h = len(ms) // 2
                        ms = [jnp.where(mk, ms[i], ms[i + h][xp])
                              + jnp.where(mk, ms[i][xp], ms[i + h])
                              for i in range(h)]
                    logit.at[t * K + k, g][pl.ds(half * 16, 16)] = ms[0]
                return 0
            lax.fori_loop(0, K, k_loop, 0)
            return 0
        lax.fori_loop(0, T * G, logits_tg, 0)
        fire(vtab, 0, 0)

        for g in range(G):
            for half in range(2):
                sl = pl.ds(half * 16, 16)
                vals = [logit[l, g, sl] for l in range(L)]
                m = vals[0]
                for l in range(1, L):
                    m = jnp.maximum(m, vals[l])
                es = [jnp.exp(v - m) for v in vals]
                ssum = es[0]
                for l in range(1, L):
                    ssum = ssum + es[l]
                inv = 1.0 / ssum
                for l in range(L):
                    logit.at[l, g][sl] = es[l] * inv

        def acc_tg(tg, _):
            t = tg // G
            g = tg - t * G
            buf = tg & 1
            drain(vtab, tg, buf)

            @pl.when(tg < T * G - 1)
            def _():
                fire(vtab, tg + 1, 1 - buf)

            def k_loop(k, _):
                wb = tg * (K * 8) + k * 8
                for half in range(2):
                    awrow = logit[t * K + k, g, pl.ds(half * 16, 16)]
                    w0 = wv[wb + half] * awrow
                    w1 = wv[wb + 2 + half] * awrow
                    w2 = wv[wb + 4 + half] * awrow
                    w3 = wv[wb + 6 + half] * awrow
                    for ln in range(16):
                        base = k * 128 + half * 16 + ln
                        spl = jnp.full((16,), ln, jnp.int32)
                        vs = (w0[spl] * rows[buf, base]
                              + w1[spl] * rows[buf, base + 32]
                              + w2[spl] * rows[buf, base + 64]
                              + w3[spl] * rows[buf, base + 96])
                        orow = (half * 16 + ln) * G + g
                        outv.at[cbuf, orow][...] = outv[cbuf, orow] + vs
                return 0
            lax.fori_loop(0, K, k_loop, 0)
            return 0
        lax.fori_loop(0, T * G, acc_tg, 0)

        pltpu.async_copy(outv.at[cbuf], out_hbm.at[pl.ds(p0 * G, PX * G)],
                         sem3)
        return 0

    lax.fori_loop(0, NCHUNK, chunk_body, 0)
    for cj in (NCHUNK - 2, NCHUNK - 1):
        pj = wid * PPW + cj * PX
        pltpu.make_async_copy(outv.at[cj & 1],
                              out_hbm.at[pl.ds(pj * G, PX * G)], sem3).wait()


def _sc_attention(ktab, vtab, qtab, offc):
    mesh = plsc.VectorSubcoreMesh(core_axis_name="c", subcore_axis_name="s")
    fn = functools.partial(
        pl.kernel,
        mesh=mesh,
        compiler_params=pltpu.CompilerParams(use_tc_tiling_on_sc=False),
        out_type=jax.ShapeDtypeStruct((HW * G, CG), jnp.float32),
        scratch_types=[
            pltpu.VMEM((2, NCH, PX), jnp.float32),
            pltpu.VMEM((2, PX * G, CG), jnp.float32),
            pltpu.VMEM((T * G, K, 128), jnp.int32),
            pltpu.VMEM((T * G * K * 8, CG), jnp.float32),
            pltpu.VMEM((2, K * 128, CG), jnp.float32),
            pltpu.VMEM((L, G, PX), jnp.float32),
            pltpu.VMEM((2, PX * G, CG), jnp.float32),
            pltpu.SemaphoreType.DMA,
            pltpu.SemaphoreType.DMA,
            pltpu.SemaphoreType.DMA,
        ],
    )(_attn_body)
    return fn(ktab, vtab, qtab, offc)


def kernel(q, k, v, offset, Wq, bq, Wk, bk, Wv, bv, W1, b1, W2, b2):
    scale = 1.0 / jnp.sqrt(jnp.float32(CG))
    x5 = jnp.concatenate([q.reshape(1, C, HW),
                          k.reshape(T, C, HW),
                          v.reshape(T, C, HW)], axis=0)
    wt5 = jnp.stack([Wq.T * scale, Wk.T, Wk.T, Wv.T, Wv.T], axis=0)
    b5 = jnp.stack([bq * scale, bk, bk, bv, bv], axis=0)
    y5 = _project(x5, wt5, b5)

    qtab = y5[0].reshape(HW * G, CG)
    ktab = y5[1:3].reshape(T * HW * G, CG)
    vtab = y5[3:5].reshape(T * HW * G, CG)
    offc = (offset.reshape(NCH, HW // PX, PX)
            .transpose(1, 0, 2))

    att = _sc_attention(ktab, vtab, qtab, offc)
    att = att.reshape(HW, C)

    y = _ffn(att, W1.T, b1, W2.T, b2)
    return y.T.reshape(B, 1, C, H, W)


# speedup vs baseline: 10.9406x; 1.0023x over previous
import functools

import jax
import jax.numpy as jnp
from jax import lax
from jax.experimental import pallas as pl
from jax.experimental.pallas import tpu as pltpu
from jax.experimental.pallas import tpu_sc as plsc

B, T, C, H, W = 1, 2, 96, 128, 128
KH, KW = 3, 3
G = 6
CG = C // G
K = KH * KW
HW = H * W
L = T * K

NW = 32
PX = 32
PPW = HW // NW
NCHUNK = PPW // PX
NCH = T * G * K * 2


def _proj_body(x_ref, w_ref, b_ref, o_ref):
    x = x_ref[0]
    w = w_ref[0]
    y = lax.dot_general(x, w, (((0,), (0,)), ((), ())),
                        preferred_element_type=jnp.float32)
    o_ref[0] = y + b_ref[0]


def _project(x5, wt5, b5):
    BK = 2048
    return pl.pallas_call(
        _proj_body,
        grid=(5, HW // BK),
        in_specs=[
            pl.BlockSpec((1, C, BK), lambda i, j: (i, 0, j)),
            pl.BlockSpec((1, C, C), lambda i, j: (i, 0, 0)),
            pl.BlockSpec((1, 1, C), lambda i, j: (i, 0, 0)),
        ],
        out_specs=pl.BlockSpec((1, BK, C), lambda i, j: (i, j, 0)),
        out_shape=jax.ShapeDtypeStruct((5, HW, C), jnp.float32),
    )(x5, wt5, b5.reshape(5, 1, C))


def _ffn_body(x_ref, w1_ref, b1_ref, w2_ref, b2_ref, o_ref):
    x = x_ref[...]
    h = jax.nn.gelu(jnp.dot(x, w1_ref[...], preferred_element_type=jnp.float32)
                    + b1_ref[...][None, :])
    o_ref[...] = (jnp.dot(h, w2_ref[...], preferred_element_type=jnp.float32)
                  + b2_ref[...][None, :] + x)


def _ffn(x, w1t, b1, w2t, b2):
    BK = 2048
    return pl.pallas_call(
        _ffn_body,
        grid=(HW // BK,),
        in_specs=[
            pl.BlockSpec((BK, C), lambda i: (i, 0)),
            pl.BlockSpec((C, 2 * C), lambda i: (0, 0)),
            pl.BlockSpec((2 * C,), lambda i: (0,)),
            pl.BlockSpec((2 * C, C), lambda i: (0, 0)),
            pl.BlockSpec((C,), lambda i: (0,)),
        ],
        out_specs=pl.BlockSpec((BK, C), lambda i: (i, 0)),
        out_shape=jax.ShapeDtypeStruct((HW, C), jnp.float32),
    )(x, w1t, b1, w2t, b2)


def _floor_f32(x):
    ti = x.astype(jnp.int32).astype(jnp.float32)
    return ti - jnp.where(x < ti, 1.0, 0.0)


def _attn_body(ktab, vtab, qtab, offc, out_hbm,
               offv, qv, idxv, wv, rows, logit, outv, sem, sem2, sem3):
    wid = lax.axis_index("s") * 2 + lax.axis_index("c")
    lanes = lax.iota(jnp.int32, 16)

    pltpu.async_copy(offc.at[wid * NCHUNK], offv.at[0], sem2)
    pltpu.async_copy(qtab.at[pl.ds(wid * PPW * G, PX * G)], qv.at[0], sem2)

    def chunk_body(ci, _):
        p0 = wid * PPW + ci * PX
        cbuf = ci & 1
        pltpu.make_async_copy(offc.at[wid * NCHUNK + ci], offv.at[cbuf],
                              sem2).wait()
        pltpu.make_async_copy(qtab.at[pl.ds(p0 * G, PX * G)], qv.at[cbuf],
                              sem2).wait()

        @pl.when(ci < NCHUNK - 1)
        def _():
            pltpu.async_copy(offc.at[wid * NCHUNK + ci + 1],
                             offv.at[1 - cbuf], sem2)
            pltpu.async_copy(qtab.at[pl.ds((p0 + PX) * G, PX * G)],
                             qv.at[1 - cbuf], sem2)

        @pl.when(ci >= 2)
        def _():
            pltpu.make_async_copy(
                outv.at[cbuf],
                out_hbm.at[pl.ds((p0 - 2 * PX) * G, PX * G)], sem3).wait()

        def zero_body(i, _):
            outv.at[cbuf, i][...] = jnp.zeros((16,), jnp.float32)
            return 0
        lax.fori_loop(0, PX * G, zero_body, 0)

        def idx_body(s, _):
            tg = s // K
            k = s - tg * K
            t = tg // G
            g = tg - t * G
            ki = k // KW
            kj = k - ki * KW
            for half in range(2):
                p_idx = p0 + half * 16 + lanes
                pyv = p_idx >> 7
                pxv = p_idx & 127
                dy = offv[cbuf, tg * (K * 2) + k * 2, pl.ds(half * 16, 16)]
                dx = offv[cbuf, tg * (K * 2) + k * 2 + 1, pl.ds(half * 16, 16)]
                sy = pyv.astype(jnp.float32) + (ki - 1).astype(jnp.float32) + dy
                sx = pxv.astype(jnp.float32) + (kj - 1).astype(jnp.float32) + dx
                sy = jnp.minimum(jnp.maximum(sy, -2.0), 130.0)
                sx = jnp.minimum(jnp.maximum(sx, -2.0), 130.0)
                y0 = _floor_f32(sy)
                x0 = _floor_f32(sx)
                wy1 = sy - y0
                wx1 = sx - x0
                for a in range(2):
                    ya = y0 + float(a)
                    vay = (ya >= 0.0) & (ya <= float(H - 1))
                    yi = jnp.minimum(jnp.maximum(ya, 0.0),
                                     float(H - 1)).astype(jnp.int32)
                    wya = wy1 if a == 1 else 1.0 - wy1
                    for bq_ in range(2):
                        xb = x0 + float(bq_)
                        vbx = (xb >= 0.0) & (xb <= float(W - 1))
                        xi = jnp.minimum(jnp.maximum(xb, 0.0),
                                         float(W - 1)).astype(jnp.int32)
                        wxb = wx1 if bq_ == 1 else 1.0 - wx1
                        c = a * 2 + bq_
                        msk = jnp.where(vay & vbx, 1.0, 0.0)
                        ridx = t * (HW * G) + ((yi << 7) + xi) * G + g
                        idxv.at[tg, k][pl.ds((c * 2 + half) * 16, 16)] = ridx
                        wv.at[tg * (K * 8) + k * 8 + c * 2 + half][...] = (
                            wya * wxb * msk)
            return 0
        lax.fori_loop(0, T * G * K, idx_body, 0)

        def fire(tab, tg, buf):
            for k in range(K):
                pltpu.async_copy(tab.at[idxv.at[tg, k]],
                                 rows.at[buf, pl.ds(k * 128, 128)], sem)

        def drain(tab, tg, buf):
            for k in range(K):
                pltpu.make_async_copy(tab.at[idxv.at[tg, k]],
                                      rows.at[buf, pl.ds(k * 128, 128)],
                                      sem).wait()

        fire(ktab, 0, 0)

        def logits_tg(tg, _):
            t = tg // G
            g = tg - t * G
            buf = tg & 1
            drain(ktab, tg, buf)

            @pl.when(tg < T * G - 1)
            def _():
                fire(ktab, tg + 1, 1 - buf)

            def k_loop(k, _):
                wb = tg * (K * 8) + k * 8
                for half in range(2):
                    w0 = wv[wb + half]
                    w1 = wv[wb + 2 + half]
                    w2 = wv[wb + 4 + half]
                    w3 = wv[wb + 6 + half]
                    ms = []
                    for ln in range(16):
                        base = k * 128 + half * 16 + ln
                        spl = jnp.full((16,), ln, jnp.int32)
                        ks = (w0[spl] * rows[buf, base]
                              + w1[spl] * rows[buf, base + 32]
                              + w2[spl] * rows[buf, base + 64]
                              + w3[spl] * rows[buf, base + 96])
                        ms.append(ks * qv[cbuf, (half * 16 + ln) * G + g])
                    for s in (8, 4, 2, 1):
                        xp = lanes ^ s
                        mk = (lanes & s) == 0
                        h = len(ms) // 2
                        ms = [jnp.where(mk, ms[i], ms[i + h][xp])
                              + jnp.where(mk, ms[i][xp], ms[i + h])
                              for i in range(h)]
                    logit.at[t * K + k, g][pl.ds(half * 16, 16)] = ms[0]
                return 0
            lax.fori_loop(0, K, k_loop, 0)
            return 0
        lax.fori_loop(0, T * G, logits_tg, 0)
        fire(vtab, 0, 0)

        for g in range(G):
            for half in range(2):
                sl = pl.ds(half * 16, 16)
                vals = [logit[l, g, sl] for l in range(L)]
                m = vals[0]
                for l in range(1, L):
                    m = jnp.maximum(m, vals[l])
                es = [jnp.exp(v - m) for v in vals]
                ssum = es[0]
                for l in range(1, L):
                    ssum = ssum + es[l]
                inv = 1.0 / ssum
                for l in range(L):
                    logit.at[l, g][sl] = es[l] * inv

        def acc_tg(tg, _):
            t = tg // G
            g = tg - t * G
            buf = tg & 1
            drain(vtab, tg, buf)

            @pl.when(tg < T * G - 1)
            def _():
                fire(vtab, tg + 1, 1 - buf)

            def k_loop(k, _):
                wb = tg * (K * 8) + k * 8
                for half in range(2):
                    awrow = logit[t * K + k, g, pl.ds(half * 16, 16)]
                    w0 = wv[wb + half] * awrow
                    w1 = wv[wb + 2 + half] * awrow
                    w2 = wv[wb + 4 + half] * awrow
                    w3 = wv[wb + 6 + half] * awrow
                    for ln in range(16):
                        base = k * 128 + half * 16 + ln
                        spl = jnp.full((16,), ln, jnp.int32)
                        vs = (w0[spl] * rows[buf, base]
                              + w1[spl] * rows[buf, base + 32]
                              + w2[spl] * rows[buf, base + 64]
                              + w3[spl] * rows[buf, base + 96])
                        orow = (half * 16 + ln) * G + g
                        outv.at[cbuf, orow][...] = outv[cbuf, orow] + vs
                return 0
            lax.fori_loop(0, K, k_loop, 0)
            return 0
        lax.fori_loop(0, T * G, acc_tg, 0)

        pltpu.async_copy(outv.at[cbuf], out_hbm.at[pl.ds(p0 * G, PX * G)],
                         sem3)
        return 0

    lax.fori_loop(0, NCHUNK, chunk_body, 0)
    for cj in (NCHUNK - 2, NCHUNK - 1):
        pj = wid * PPW + cj * PX
        pltpu.make_async_copy(outv.at[cj & 1],
                              out_hbm.at[pl.ds(pj * G, PX * G)], sem3).wait()


def _sc_attention(ktab, vtab, qtab, offc):
    mesh = plsc.VectorSubcoreMesh(core_axis_name="c", subcore_axis_name="s")
    fn = functools.partial(
        pl.kernel,
        mesh=mesh,
        compiler_params=pltpu.CompilerParams(use_tc_tiling_on_sc=False),
        out_type=jax.ShapeDtypeStruct((HW * G, CG), jnp.float32),
        scratch_types=[
            pltpu.VMEM((2, NCH, PX), jnp.float32),
            pltpu.VMEM((2, PX * G, CG), jnp.float32),
            pltpu.VMEM((T * G, K, 128), jnp.int32),
            pltpu.VMEM((T * G * K * 8, CG), jnp.float32),
            pltpu.VMEM((2, K * 128, CG), jnp.float32),
            pltpu.VMEM((L, G, PX), jnp.float32),
            pltpu.VMEM((2, PX * G, CG), jnp.float32),
            pltpu.SemaphoreType.DMA,
            pltpu.SemaphoreType.DMA,
            pltpu.SemaphoreType.DMA,
        ],
    )(_attn_body)
    return fn(ktab, vtab, qtab, offc)


def kernel(q, k, v, offset, Wq, bq, Wk, bk, Wv, bv, W1, b1, W2, b2):
    scale = 1.0 / jnp.sqrt(jnp.float32(CG))
    x5 = jnp.concatenate([q.reshape(1, C, HW),
                          k.reshape(T, C, HW),
                          v.reshape(T, C, HW)], axis=0)
    wt5 = jnp.stack([Wq.T * scale, Wk.T, Wk.T, Wv.T, Wv.T], axis=0)
    b5 = jnp.stack([bq * scale, bk, bk, bv, bv], axis=0)
    y5 = _project(x5, wt5, b5)

    qtab = y5[0].reshape(HW * G, CG)
    ktab = y5[1:3].reshape(T * HW * G, CG)
    vtab = y5[3:5].reshape(T * HW * G, CG)
    offc = (offset.reshape(NCH, HW // PX, PX)
            .transpose(1, 0, 2))

    att = _sc_attention(ktab, vtab, qtab, offc)
    att = att.reshape(HW, C)

    y = _ffn(att, W1.T, b1, W2.T, b2)
    return y.T.reshape(B, 1, C, H, W)


# speedup vs baseline: 11.3638x; 1.0387x over previous
import functools

import jax
import jax.numpy as jnp
from jax import lax
from jax.experimental import pallas as pl
from jax.experimental.pallas import tpu as pltpu
from jax.experimental.pallas import tpu_sc as plsc

B, T, C, H, W = 1, 2, 96, 128, 128
KH, KW = 3, 3
G = 6
CG = C // G
K = KH * KW
HW = H * W
L = T * K

NW = 32
PX = 32
PPW = HW // NW
NCHUNK = PPW // PX
NCH = T * G * K * 2


def _proj_body(x_ref, w_ref, b_ref, o_ref):
    x = x_ref[0]
    w = w_ref[0]
    y = lax.dot_general(x, w, (((0,), (0,)), ((), ())),
                        preferred_element_type=jnp.float32)
    o_ref[0] = y + b_ref[0]


def _project(x5, wt5, b5):
    BK = 2048
    return pl.pallas_call(
        _proj_body,
        grid=(5, HW // BK),
        in_specs=[
            pl.BlockSpec((1, C, BK), lambda i, j: (i, 0, j)),
            pl.BlockSpec((1, C, C), lambda i, j: (i, 0, 0)),
            pl.BlockSpec((1, 1, C), lambda i, j: (i, 0, 0)),
        ],
        out_specs=pl.BlockSpec((1, BK, C), lambda i, j: (i, j, 0)),
        out_shape=jax.ShapeDtypeStruct((5, HW, C), jnp.float32),
    )(x5, wt5, b5.reshape(5, 1, C))


def _ffn_body(x_ref, w1_ref, b1_ref, w2_ref, b2_ref, o_ref):
    x = x_ref[...]
    h = jax.nn.gelu(jnp.dot(x, w1_ref[...], preferred_element_type=jnp.float32)
                    + b1_ref[...][None, :])
    y = (jnp.dot(h, w2_ref[...], preferred_element_type=jnp.float32)
         + b2_ref[...][None, :] + x)
    o_ref[...] = y.T


def _ffn(x, w1t, b1, w2t, b2):
    BK = 2048
    return pl.pallas_call(
        _ffn_body,
        grid=(HW // BK,),
        in_specs=[
            pl.BlockSpec((BK, C), lambda i: (i, 0)),
            pl.BlockSpec((C, 2 * C), lambda i: (0, 0)),
            pl.BlockSpec((2 * C,), lambda i: (0,)),
            pl.BlockSpec((2 * C, C), lambda i: (0, 0)),
            pl.BlockSpec((C,), lambda i: (0,)),
        ],
        out_specs=pl.BlockSpec((C, BK), lambda i: (0, i)),
        out_shape=jax.ShapeDtypeStruct((C, HW), jnp.float32),
    )(x, w1t, b1, w2t, b2)


def _floor_f32(x):
    ti = x.astype(jnp.int32).astype(jnp.float32)
    return ti - jnp.where(x < ti, 1.0, 0.0)


def _attn_body(ktab, vtab, qtab, offc, out_hbm,
               offv, qv, idxv, wv, rows, logit, outv, sem, sem2, sem3):
    wid = lax.axis_index("s") * 2 + lax.axis_index("c")
    lanes = lax.iota(jnp.int32, 16)

    pltpu.async_copy(offc.at[:, pl.ds(wid * PPW, PX)], offv.at[0], sem2)
    pltpu.async_copy(qtab.at[pl.ds(wid * PPW * G, PX * G)], qv.at[0], sem2)

    def chunk_body(ci, _):
        p0 = wid * PPW + ci * PX
        cbuf = ci & 1
        pltpu.make_async_copy(offc.at[:, pl.ds(p0, PX)], offv.at[cbuf],
                              sem2).wait()
        pltpu.make_async_copy(qtab.at[pl.ds(p0 * G, PX * G)], qv.at[cbuf],
                              sem2).wait()

        @pl.when(ci < NCHUNK - 1)
        def _():
            pltpu.async_copy(offc.at[:, pl.ds(p0 + PX, PX)],
                             offv.at[1 - cbuf], sem2)
            pltpu.async_copy(qtab.at[pl.ds((p0 + PX) * G, PX * G)],
                             qv.at[1 - cbuf], sem2)

        @pl.when(ci >= 2)
        def _():
            pltpu.make_async_copy(
                outv.at[cbuf],
                out_hbm.at[pl.ds((p0 - 2 * PX) * G, PX * G)], sem3).wait()

        def zero_body(i, _):
            outv.at[cbuf, i][...] = jnp.zeros((16,), jnp.float32)
            return 0
        lax.fori_loop(0, PX * G, zero_body, 0)

        def idx_body(s, _):
            tg = s // K
            k = s - tg * K
            t = tg // G
            g = tg - t * G
            ki = k // KW
            kj = k - ki * KW
            for half in range(2):
                p_idx = p0 + half * 16 + lanes
                pyv = p_idx >> 7
                pxv = p_idx & 127
                dy = offv[cbuf, tg * (K * 2) + k * 2, pl.ds(half * 16, 16)]
                dx = offv[cbuf, tg * (K * 2) + k * 2 + 1, pl.ds(half * 16, 16)]
                sy = pyv.astype(jnp.float32) + (ki - 1).astype(jnp.float32) + dy
                sx = pxv.astype(jnp.float32) + (kj - 1).astype(jnp.float32) + dx
                sy = jnp.minimum(jnp.maximum(sy, -2.0), 130.0)
                sx = jnp.minimum(jnp.maximum(sx, -2.0), 130.0)
                y0 = _floor_f32(sy)
                x0 = _floor_f32(sx)
                wy1 = sy - y0
                wx1 = sx - x0
                for a in range(2):
                    ya = y0 + float(a)
                    vay = (ya >= 0.0) & (ya <= float(H - 1))
                    yi = jnp.minimum(jnp.maximum(ya, 0.0),
                                     float(H - 1)).astype(jnp.int32)
                    wya = wy1 if a == 1 else 1.0 - wy1
                    for bq_ in range(2):
                        xb = x0 + float(bq_)
                        vbx = (xb >= 0.0) & (xb <= float(W - 1))
                        xi = jnp.minimum(jnp.maximum(xb, 0.0),
                                         float(W - 1)).astype(jnp.int32)
                        wxb = wx1 if bq_ == 1 else 1.0 - wx1
                        c = a * 2 + bq_
                        msk = jnp.where(vay & vbx, 1.0, 0.0)
                        ridx = t * (HW * G) + ((yi << 7) + xi) * G + g
                        idxv.at[tg, k][pl.ds((c * 2 + half) * 16, 16)] = ridx
                        wv.at[tg * (K * 8) + k * 8 + c * 2 + half][...] = (
                            wya * wxb * msk)
            return 0
        lax.fori_loop(0, T * G * K, idx_body, 0)

        def fire(tab, tg, buf):
            for k in range(K):
                pltpu.async_copy(tab.at[idxv.at[tg, k]],
                                 rows.at[buf, pl.ds(k * 128, 128)], sem)

        def drain(tab, tg, buf):
            for k in range(K):
                pltpu.make_async_copy(tab.at[idxv.at[tg, k]],
                                      rows.at[buf, pl.ds(k * 128, 128)],
                                      sem).wait()

        fire(ktab, 0, 0)

        def logits_tg(tg, _):
            t = tg // G
            g = tg - t * G
            buf = tg & 1
            drain(ktab, tg, buf)

            @pl.when(tg < T * G - 1)
            def _():
                fire(ktab, tg + 1, 1 - buf)

            def k_loop(k, _):
                wb = tg * (K * 8) + k * 8
                for half in range(2):
                    w0 = wv[wb + half]
                    w1 = wv[wb + 2 + half]
                    w2 = wv[wb + 4 + half]
                    w3 = wv[wb + 6 + half]
                    ms = []
                    for ln in range(16):
                        base = k * 128 + half * 16 + ln
                        spl = jnp.full((16,), ln, jnp.int32)
                        ks = (w0[spl] * rows[buf, base]
                              + w1[spl] * rows[buf, base + 32]
                              + w2[spl] * rows[buf, base + 64]
                              + w3[spl] * rows[buf, base + 96])
                        ms.append(ks * qv[cbuf, (half * 16 + ln) * G + g])
                    for s in (8, 4, 2, 1):
                        xp = lanes ^ s
                        mk = (lanes & s) == 0
                        h = len(ms) // 2
                        ms = [jnp.where(mk, ms[i], ms[i + h][xp])
                              + jnp.where(mk, ms[i][xp], ms[i + h])
                              for i in range(h)]
                    logit.at[t * K + k, g][pl.ds(half * 16, 16)] = ms[0]
                return 0
            lax.fori_loop(0, K, k_loop, 0)
            return 0
        lax.fori_loop(0, T * G, logits_tg, 0)
        fire(vtab, 0, 0)

        for g in range(G):
            for half in range(2):
                sl = pl.ds(half * 16, 16)
                vals = [logit[l, g, sl] for l in range(L)]
                m = vals[0]
                for l in range(1, L):
                    m = jnp.maximum(m, vals[l])
                es = [jnp.exp(v - m) for v in vals]
                ssum = es[0]
                for l in range(1, L):
                    ssum = ssum + es[l]
                inv = 1.0 / ssum
                for l in range(L):
                    logit.at[l, g][sl] = es[l] * inv

        def acc_tg(tg, _):
            t = tg // G
            g = tg - t * G
            buf = tg & 1
            drain(vtab, tg, buf)

            @pl.when(tg < T * G - 1)
            def _():
                fire(vtab, tg + 1, 1 - buf)

            def k_loop(k, _):
                wb = tg * (K * 8) + k * 8
                for half in range(2):
                    awrow = logit[t * K + k, g, pl.ds(half * 16, 16)]
                    w0 = wv[wb + half] * awrow
                    w1 = wv[wb + 2 + half] * awrow
                    w2 = wv[wb + 4 + half] * awrow
                    w3 = wv[wb + 6 + half] * awrow
                    for ln in range(16):
                        base = k * 128 + half * 16 + ln
                        spl = jnp.full((16,), ln, jnp.int32)
                        vs = (w0[spl] * rows[buf, base]
                              + w1[spl] * rows[buf, base + 32]
                              + w2[spl] * rows[buf, base + 64]
                              + w3[spl] * rows[buf, base + 96])
                        orow = (half * 16 + ln) * G + g
                        outv.at[cbuf, orow][...] = outv[cbuf, orow] + vs
                return 0
            lax.fori_loop(0, K, k_loop, 0)
            return 0
        lax.fori_loop(0, T * G, acc_tg, 0)

        pltpu.async_copy(outv.at[cbuf], out_hbm.at[pl.ds(p0 * G, PX * G)],
                         sem3)
        return 0

    lax.fori_loop(0, NCHUNK, chunk_body, 0)
    for cj in (NCHUNK - 2, NCHUNK - 1):
        pj = wid * PPW + cj * PX
        pltpu.make_async_copy(outv.at[cj & 1],
                              out_hbm.at[pl.ds(pj * G, PX * G)], sem3).wait()


def _sc_attention(ktab, vtab, qtab, offc):
    mesh = plsc.VectorSubcoreMesh(core_axis_name="c", subcore_axis_name="s")
    fn = functools.partial(
        pl.kernel,
        mesh=mesh,
        compiler_params=pltpu.CompilerParams(use_tc_tiling_on_sc=False),
        out_type=jax.ShapeDtypeStruct((HW * G, CG), jnp.float32),
        scratch_types=[
            pltpu.VMEM((2, NCH, PX), jnp.float32),
            pltpu.VMEM((2, PX * G, CG), jnp.float32),
            pltpu.VMEM((T * G, K, 128), jnp.int32),
            pltpu.VMEM((T * G * K * 8, CG), jnp.float32),
            pltpu.VMEM((2, K * 128, CG), jnp.float32),
            pltpu.VMEM((L, G, PX), jnp.float32),
            pltpu.VMEM((2, PX * G, CG), jnp.float32),
            pltpu.SemaphoreType.DMA,
            pltpu.SemaphoreType.DMA,
            pltpu.SemaphoreType.DMA,
        ],
    )(_attn_body)
    return fn(ktab, vtab, qtab, offc)


def kernel(q, k, v, offset, Wq, bq, Wk, bk, Wv, bv, W1, b1, W2, b2):
    scale = 1.0 / jnp.sqrt(jnp.float32(CG))
    x5 = jnp.concatenate([q.reshape(1, C, HW),
                          k.reshape(T, C, HW),
                          v.reshape(T, C, HW)], axis=0)
    wt5 = jnp.stack([Wq.T * scale, Wk.T, Wk.T, Wv.T, Wv.T], axis=0)
    b5 = jnp.stack([bq * scale, bk, bk, bv, bv], axis=0)
    y5 = _project(x5, wt5, b5)

    qtab = y5[0].reshape(HW * G, CG)
    ktab = y5[1:3].reshape(T * HW * G, CG)
    vtab = y5[3:5].reshape(T * HW * G, CG)
    offc = offset.reshape(NCH, HW)

    att = _sc_attention(ktab, vtab, qtab, offc)
    att = att.reshape(HW, C)

    y = _ffn(att, W1.T, b1, W2.T, b2)
    return y.reshape(B, 1, C, H, W)
